# baseline (speedup 1.0000x reference)
import functools

import jax
import jax.numpy as jnp
from jax import lax
from jax.experimental import pallas as pl
from jax.experimental.pallas import tpu as pltpu
from jax.experimental.pallas import tpu_sc as plsc

B = 16384
D = 128
L = 16
NC = 2
NS = 16
NW = NC * NS
BPW = B // NW
CHUNK = 64
NCHUNK = BPW // CHUNK
NBUF = 4

_BITREV = [0, 8, 4, 12, 2, 10, 6, 14, 1, 9, 5, 13, 3, 11, 7, 15]


def _lane_shuffle(x, perm):
    dnums = lax.GatherDimensionNumbers(
        offset_dims=(), collapsed_slice_dims=(0,), start_index_map=(0,))
    return lax.gather(x, perm[:, None], dnums, (1,),
                      mode=lax.GatherScatterMode.PROMISE_IN_BOUNDS)


@functools.cache
def _make_kernel():
    mesh = plsc.VectorSubcoreMesh(core_axis_name="c", subcore_axis_name="s",
                                  num_cores=NC)

    @functools.partial(
        pl.kernel,
        mesh=mesh,
        out_type=jax.ShapeDtypeStruct((B,), jnp.float32),
        scratch_types=(
            [pltpu.VMEM((BPW,), jnp.int32)] * 2
            + [pltpu.VMEM((CHUNK, D), jnp.float32)] * (2 * NBUF)
            + [pltpu.VMEM((BPW,), jnp.float32)]
            + [pltpu.SemaphoreType.DMA] * (2 * NBUF + 2)
        ),
    )
    def k(user_hbm, item_hbm, utab_hbm, itab_hbm, out_hbm, *refs):
        uidx_v, iidx_v = refs[0], refs[1]
        ubufs = refs[2:2 + NBUF]
        ibufs = refs[2 + NBUF:2 + 2 * NBUF]
        out_v = refs[2 + 2 * NBUF]
        usems = refs[3 + 2 * NBUF:3 + 3 * NBUF]
        isems = refs[3 + 3 * NBUF:3 + 4 * NBUF]
        sidx_u, sidx_i = refs[3 + 4 * NBUF], refs[4 + 4 * NBUF]

        wid = lax.axis_index("s") * NC + lax.axis_index("c")
        base = wid * BPW

        cu = pltpu.async_copy(user_hbm.at[pl.ds(base, BPW)], uidx_v, sidx_u)
        ci = pltpu.async_copy(item_hbm.at[pl.ds(base, BPW)], iidx_v, sidx_i)
        cu.wait()
        ci.wait()

        def start(c):
            s = c % NBUF
            cu = pltpu.async_copy(
                utab_hbm.at[uidx_v.at[pl.ds(c * CHUNK, CHUNK)]],
                ubufs[s], usems[s])
            ci = pltpu.async_copy(
                itab_hbm.at[iidx_v.at[pl.ds(c * CHUNK, CHUNK)]],
                ibufs[s], isems[s])
            return cu, ci

        def merge(a, b, sh, lane):
            m = (lane & sh) == 0
            a_s = _lane_shuffle(a, lane ^ sh)
            b_s = _lane_shuffle(b, lane ^ sh)
            return jnp.where(m, a, b_s) + jnp.where(m, a_s, b)

        pend = [start(c) for c in range(min(NBUF, NCHUNK))]
        for c in range(NCHUNK):
            cu, ci = pend[c % NBUF]
            cu.wait()
            ci.wait()
            ub = ubufs[c % NBUF]
            ib = ibufs[c % NBUF]

            def grp_body(g, _):
                lane = lax.iota(jnp.int32, L)
                stack = []
                for i in range(L):
                    r = g * L + _BITREV[i]
                    acc = jnp.zeros((L,), jnp.float32)
                    for q in range(D // L):
                        u = ub[r, pl.ds(q * L, L)]
                        v = ib[r, pl.ds(q * L, L)]
                        acc = acc + u * v
                    node = (0, acc)
                    while stack and stack[-1][0] == node[0]:
                        lvl, prev = stack.pop()
                        node = (lvl + 1, merge(prev, node[1], 8 >> lvl, lane))
                    stack.append(node)
                res = stack[0][1]
                out_v[pl.ds(c * CHUNK + g * L, L)] = res
                return 0

            lax.fori_loop(0, CHUNK // L, grp_body, 0, unroll=False)

            if c + NBUF < NCHUNK:
                pend[c % NBUF] = start(c + NBUF)

        pltpu.sync_copy(out_v, out_hbm.at[pl.ds(base, BPW)])

    return k


def kernel(user, item, user_table, item_table, training=0):
    del training
    return _make_kernel()(user.astype(jnp.int32), item.astype(jnp.int32),
                          user_table, item_table)


# speedup vs baseline: 1.2363x; 1.2363x over previous
import functools

import jax
import jax.numpy as jnp
from jax import lax
from jax.experimental import pallas as pl
from jax.experimental.pallas import tpu as pltpu
from jax.experimental.pallas import tpu_sc as plsc

B = 16384
D = 128
L = 16
NC = 2
NS = 16
NW = NC * NS
BPW = B // NW
CHUNK = 128
NCHUNK = BPW // CHUNK
NBUF = 3

_BITREV = [0, 8, 4, 12, 2, 10, 6, 14, 1, 9, 5, 13, 3, 11, 7, 15]


def _lane_shuffle(x, perm):
    dnums = lax.GatherDimensionNumbers(
        offset_dims=(), collapsed_slice_dims=(0,), start_index_map=(0,))
    return lax.gather(x, perm[:, None], dnums, (1,),
                      mode=lax.GatherScatterMode.PROMISE_IN_BOUNDS)


@functools.cache
def _make_kernel():
    mesh = plsc.VectorSubcoreMesh(core_axis_name="c", subcore_axis_name="s",
                                  num_cores=NC)

    @functools.partial(
        pl.kernel,
        mesh=mesh,
        out_type=jax.ShapeDtypeStruct((B,), jnp.float32),
        scratch_types=(
            [pltpu.VMEM((BPW,), jnp.int32)] * 2
            + [pltpu.VMEM((CHUNK, D), jnp.float32)] * (2 * NBUF)
            + [pltpu.VMEM((BPW,), jnp.float32)]
            + [pltpu.SemaphoreType.DMA] * (2 * NBUF + 2)
        ),
    )
    def k(user_hbm, item_hbm, utab_hbm, itab_hbm, out_hbm, *refs):
        uidx_v, iidx_v = refs[0], refs[1]
        ubufs = refs[2:2 + NBUF]
        ibufs = refs[2 + NBUF:2 + 2 * NBUF]
        out_v = refs[2 + 2 * NBUF]
        usems = refs[3 + 2 * NBUF:3 + 3 * NBUF]
        isems = refs[3 + 3 * NBUF:3 + 4 * NBUF]
        sidx_u, sidx_i = refs[3 + 4 * NBUF], refs[4 + 4 * NBUF]

        wid = lax.axis_index("s") * NC + lax.axis_index("c")
        base = wid * BPW

        cu = pltpu.async_copy(user_hbm.at[pl.ds(base, BPW)], uidx_v, sidx_u)
        ci = pltpu.async_copy(item_hbm.at[pl.ds(base, BPW)], iidx_v, sidx_i)
        cu.wait()
        ci.wait()

        def start(c):
            s = c % NBUF
            cu = pltpu.async_copy(
                utab_hbm.at[uidx_v.at[pl.ds(c * CHUNK, CHUNK)]],
                ubufs[s], usems[s])
            ci = pltpu.async_copy(
                itab_hbm.at[iidx_v.at[pl.ds(c * CHUNK, CHUNK)]],
                ibufs[s], isems[s])
            return cu, ci

        def merge(a, b, sh, lane):
            m = (lane & sh) == 0
            a_s = _lane_shuffle(a, lane ^ sh)
            b_s = _lane_shuffle(b, lane ^ sh)
            return jnp.where(m, a, b_s) + jnp.where(m, a_s, b)

        pend = [start(c) for c in range(min(NBUF, NCHUNK))]
        for c in range(NCHUNK):
            cu, ci = pend[c % NBUF]
            cu.wait()
            ci.wait()
            ub = ubufs[c % NBUF]
            ib = ibufs[c % NBUF]

            def grp_body(g, _):
                lane = lax.iota(jnp.int32, L)
                stack = []
                for i in range(L):
                    r = g * L + _BITREV[i]
                    acc = jnp.zeros((L,), jnp.float32)
                    for q in range(D // L):
                        u = ub[r, pl.ds(q * L, L)]
                        v = ib[r, pl.ds(q * L, L)]
                        acc = acc + u * v
                    node = (0, acc)
                    while stack and stack[-1][0] == node[0]:
                        lvl, prev = stack.pop()
                        node = (lvl + 1, merge(prev, node[1], 8 >> lvl, lane))
                    stack.append(node)
                res = stack[0][1]
                out_v[pl.ds(c * CHUNK + g * L, L)] = res
                return 0

            lax.fori_loop(0, CHUNK // L, grp_body, 0, unroll=False)

            if c + NBUF < NCHUNK:
                pend[c % NBUF] = start(c + NBUF)

        pltpu.sync_copy(out_v, out_hbm.at[pl.ds(base, BPW)])

    return k


def kernel(user, item, user_table, item_table, training=0):
    del training
    return _make_kernel()(user.astype(jnp.int32), item.astype(jnp.int32),
                          user_table, item_table)


# speedup vs baseline: 1.2631x; 1.0217x over previous
import functools

import jax
import jax.numpy as jnp
from jax import lax
from jax.experimental import pallas as pl
from jax.experimental.pallas import tpu as pltpu
from jax.experimental.pallas import tpu_sc as plsc

B = 16384
D = 128
L = 16
NC = 2
NS = 16
NW = NC * NS
BPW = B // NW
CHUNK = 128
NCHUNK = BPW // CHUNK
NBUF = 2

_BITREV = [0, 8, 4, 12, 2, 10, 6, 14, 1, 9, 5, 13, 3, 11, 7, 15]


def _lane_shuffle(x, perm):
    dnums = lax.GatherDimensionNumbers(
        offset_dims=(), collapsed_slice_dims=(0,), start_index_map=(0,))
    return lax.gather(x, perm[:, None], dnums, (1,),
                      mode=lax.GatherScatterMode.PROMISE_IN_BOUNDS)


@functools.cache
def _make_kernel():
    mesh = plsc.VectorSubcoreMesh(core_axis_name="c", subcore_axis_name="s",
                                  num_cores=NC)

    @functools.partial(
        pl.kernel,
        mesh=mesh,
        out_type=jax.ShapeDtypeStruct((B,), jnp.float32),
        scratch_types=(
            [pltpu.VMEM((BPW,), jnp.int32)] * 2
            + [pltpu.VMEM((CHUNK, D), jnp.float32)] * (2 * NBUF)
            + [pltpu.VMEM((BPW,), jnp.float32)]
            + [pltpu.SemaphoreType.DMA] * (2 * NBUF + 2)
        ),
    )
    def k(user_hbm, item_hbm, utab_hbm, itab_hbm, out_hbm, *refs):
        uidx_v, iidx_v = refs[0], refs[1]
        ubufs = refs[2:2 + NBUF]
        ibufs = refs[2 + NBUF:2 + 2 * NBUF]
        out_v = refs[2 + 2 * NBUF]
        usems = refs[3 + 2 * NBUF:3 + 3 * NBUF]
        isems = refs[3 + 3 * NBUF:3 + 4 * NBUF]
        sidx_u, sidx_i = refs[3 + 4 * NBUF], refs[4 + 4 * NBUF]

        wid = lax.axis_index("s") * NC + lax.axis_index("c")
        base = wid * BPW

        cu = pltpu.async_copy(user_hbm.at[pl.ds(base, BPW)], uidx_v, sidx_u)
        ci = pltpu.async_copy(item_hbm.at[pl.ds(base, BPW)], iidx_v, sidx_i)
        cu.wait()
        ci.wait()

        def start(c):
            s = c % NBUF
            cu = pltpu.async_copy(
                utab_hbm.at[uidx_v.at[pl.ds(c * CHUNK, CHUNK)]],
                ubufs[s], usems[s])
            ci = pltpu.async_copy(
                itab_hbm.at[iidx_v.at[pl.ds(c * CHUNK, CHUNK)]],
                ibufs[s], isems[s])
            return cu, ci

        def merge(a, b, sh, lane):
            m = (lane & sh) == 0
            a_s = _lane_shuffle(a, lane ^ sh)
            b_s = _lane_shuffle(b, lane ^ sh)
            return jnp.where(m, a, b_s) + jnp.where(m, a_s, b)

        pend = [start(c) for c in range(min(NBUF, NCHUNK))]
        for c in range(NCHUNK):
            cu, ci = pend[c % NBUF]
            cu.wait()
            ci.wait()
            ub = ubufs[c % NBUF]
            ib = ibufs[c % NBUF]

            def grp_body(g, _):
                lane = lax.iota(jnp.int32, L)
                stack = []
                for i in range(L):
                    r = g * L + _BITREV[i]
                    acc = jnp.zeros((L,), jnp.float32)
                    for q in range(D // L):
                        u = ub[r, pl.ds(q * L, L)]
                        v = ib[r, pl.ds(q * L, L)]
                        acc = acc + u * v
                    node = (0, acc)
                    while stack and stack[-1][0] == node[0]:
                        lvl, prev = stack.pop()
                        node = (lvl + 1, merge(prev, node[1], 8 >> lvl, lane))
                    stack.append(node)
                res = stack[0][1]
                out_v[pl.ds(c * CHUNK + g * L, L)] = res
                return 0

            lax.fori_loop(0, CHUNK // L, grp_body, 0, unroll=False)

            if c + NBUF < NCHUNK:
                pend[c % NBUF] = start(c + NBUF)

        pltpu.sync_copy(out_v, out_hbm.at[pl.ds(base, BPW)])

    return k


def kernel(user, item, user_table, item_table, training=0):
    del training
    return _make_kernel()(user.astype(jnp.int32), item.astype(jnp.int32),
                          user_table, item_table)


# speedup vs baseline: 1.3109x; 1.0378x over previous
import functools

import jax
import jax.numpy as jnp
from jax import lax
from jax.experimental import pallas as pl
from jax.experimental.pallas import tpu as pltpu
from jax.experimental.pallas import tpu_sc as plsc

B = 16384
D = 128
L = 16
NC = 2
NS = 16
NW = NC * NS
BPW = B // NW
CHUNK = 128
NCHUNK = BPW // CHUNK
NBUF = 2

_BITREV = [0, 8, 4, 12, 2, 10, 6, 14, 1, 9, 5, 13, 3, 11, 7, 15]


def _lane_shuffle(x, perm):
    dnums = lax.GatherDimensionNumbers(
        offset_dims=(), collapsed_slice_dims=(0,), start_index_map=(0,))
    return lax.gather(x, perm[:, None], dnums, (1,),
                      mode=lax.GatherScatterMode.PROMISE_IN_BOUNDS)


@functools.cache
def _make_kernel():
    mesh = plsc.VectorSubcoreMesh(core_axis_name="c", subcore_axis_name="s",
                                  num_cores=NC)

    @functools.partial(
        pl.kernel,
        mesh=mesh,
        out_type=jax.ShapeDtypeStruct((B,), jnp.float32),
        scratch_types=(
            [pltpu.VMEM((BPW,), jnp.int32)] * 2
            + [pltpu.VMEM((CHUNK, D), jnp.float32)] * (2 * NBUF)
            + [pltpu.VMEM((BPW,), jnp.float32)]
            + [pltpu.SemaphoreType.DMA] * (2 * NBUF + 2)
        ),
    )
    def k(user_hbm, item_hbm, utab_hbm, itab_hbm, out_hbm, *refs):
        uidx_v, iidx_v = refs[0], refs[1]
        ubufs = refs[2:2 + NBUF]
        ibufs = refs[2 + NBUF:2 + 2 * NBUF]
        out_v = refs[2 + 2 * NBUF]
        usems = refs[3 + 2 * NBUF:3 + 3 * NBUF]
        isems = refs[3 + 3 * NBUF:3 + 4 * NBUF]
        sidx_u, sidx_i = refs[3 + 4 * NBUF], refs[4 + 4 * NBUF]

        wid = lax.axis_index("s") * NC + lax.axis_index("c")
        base = wid * BPW

        cu = pltpu.async_copy(user_hbm.at[pl.ds(base, BPW)], uidx_v, sidx_u)
        ci = pltpu.async_copy(item_hbm.at[pl.ds(base, BPW)], iidx_v, sidx_i)
        cu.wait()
        ci.wait()

        def start(c):
            s = c % NBUF
            cu = pltpu.async_copy(
                utab_hbm.at[uidx_v.at[pl.ds(c * CHUNK, CHUNK)]],
                ubufs[s], usems[s])
            ci = pltpu.async_copy(
                itab_hbm.at[iidx_v.at[pl.ds(c * CHUNK, CHUNK)]],
                ibufs[s], isems[s])
            return cu, ci

        def merge(a, b, sh, lane):
            m = (lane & sh) == 0
            a_s = _lane_shuffle(a, lane ^ sh)
            b_s = _lane_shuffle(b, lane ^ sh)
            return jnp.where(m, a, b_s) + jnp.where(m, a_s, b)

        pend = [None] * NBUF
        pend[0] = start(0)
        for c in range(NCHUNK):
            cu, ci = pend[c % NBUF]
            cu.wait()
            ci.wait()
            if c + 1 < NCHUNK:
                pend[(c + 1) % NBUF] = start(c + 1)
            ub = ubufs[c % NBUF]
            ib = ibufs[c % NBUF]

            def grp_body(g, _):
                lane = lax.iota(jnp.int32, L)
                stack = []
                for i in range(L):
                    r = g * L + _BITREV[i]
                    acc = jnp.zeros((L,), jnp.float32)
                    for q in range(D // L):
                        u = ub[r, pl.ds(q * L, L)]
                        v = ib[r, pl.ds(q * L, L)]
                        acc = acc + u * v
                    node = (0, acc)
                    while stack and stack[-1][0] == node[0]:
                        lvl, prev = stack.pop()
                        node = (lvl + 1, merge(prev, node[1], 8 >> lvl, lane))
                    stack.append(node)
                res = stack[0][1]
                out_v[pl.ds(c * CHUNK + g * L, L)] = res
                return 0

            lax.fori_loop(0, CHUNK // L, grp_body, 0, unroll=False)

        pltpu.sync_copy(out_v, out_hbm.at[pl.ds(base, BPW)])

    return k


def kernel(user, item, user_table, item_table, training=0):
    del training
    return _make_kernel()(user.astype(jnp.int32), item.astype(jnp.int32),
                          user_table, item_table)
